# baseline (speedup 1.0000x reference)
import functools

import jax
import jax.numpy as jnp
from jax import lax
from jax.experimental import pallas as pl
from jax.experimental.pallas import tpu as pltpu
from jax.experimental.pallas import tpu_sc as plsc

B, L, K = 1024, 200, 3
D = 64
V = 1000518
NC, NS = 2, 16
NW = NC * NS
B_PER_W = B // NW


CB = 1024
G = -(-V // CB)
VP = G * CB


TCOLS = -(-V // 128)
TPW = -(-TCOLS // NW)
TAIL_TC = TCOLS - 1
TAIL_W = V - TAIL_TC * 128
PKR = TCOLS * D


def _sc_detile(weT):
    mesh = plsc.VectorSubcoreMesh(core_axis_name="c", subcore_axis_name="s")

    @functools.partial(
        pl.kernel,
        mesh=mesh,
        out_type=jax.ShapeDtypeStruct((PKR, 2 * D), jnp.float32),
        scratch_types=[
            pltpu.VMEM((D, 2 * D), jnp.float32),
            pltpu.VMEM((D, 2 * D), jnp.float32),
            pltpu.VMEM((D, 2 * D), jnp.float32),
            pltpu.VMEM((D, 2 * D), jnp.float32),
            pltpu.SemaphoreType.DMA,
            pltpu.SemaphoreType.DMA,
            pltpu.SemaphoreType.DMA,
            pltpu.SemaphoreType.DMA,
        ],
        compiler_params=pltpu.CompilerParams(
            use_tc_tiling_on_sc=True, needs_layout_passes=False),
    )
    def k(weT_hbm, tailT_hbm, out_hbm, vin0, vin1, vout0, vout1,
          gs0, gs1, ws0, ws1):
        vin = (vin0, vin1)
        vout = (vout0, vout1)
        gs = (gs0, gs1)
        ws = (ws0, ws1)

        wid = lax.axis_index("s") * NC + lax.axis_index("c")
        t0 = wid * TPW
        iot = lax.broadcasted_iota(jnp.int32, (16,), 0)

        def in_copy(tc, b):
            return pltpu.make_async_copy(
                weT_hbm.at[:, pl.ds(tc * 128, 128)], vin[b], gs[b])

        def out_copy(tc, b):
            return pltpu.make_async_copy(
                vout[b], out_hbm.at[pl.ds(tc * D, D)], ws[b])

        def transpose_block(vi, vo, npairs):
            def pbody(p, _):
                for v in range(8):
                    d0 = 16 * (v % 4)
                    c = 2 * p + v // 4
                    vals = plsc.load_gather(vi, [iot + d0, iot * 0 + c])
                    vo[p, pl.ds(16 * v, 16)] = vals
                return 0

            lax.fori_loop(0, npairs, pbody, 0)

        def valid(g):
            return t0 + g < TAIL_TC

        @pl.when(valid(0))
        def _():
            in_copy(t0, 0).start()

        def step(g, b, prefetch=True):
            if prefetch:
                @pl.when(valid(g + 1))
                def _():
                    in_copy(t0 + g + 1, 1 - b).start()

            @pl.when((g >= 2) & valid(g - 2))
            def _():
                out_copy(t0 + g - 2, b).wait()

            @pl.when(valid(g))
            def _():
                in_copy(t0 + g, b).wait()
                transpose_block(vin[b], vout[b], D)
                out_copy(t0 + g, b).start()

        def pair_body(h, _):
            step(2 * h, 0)
            step(2 * h + 1, 1)
            return 0

        lax.fori_loop(0, TPW // 2, pair_body, 0)
        step(TPW - 1, (TPW - 1) % 2, prefetch=False)

        @pl.when(valid(TPW - 2))
        def _():
            out_copy(t0 + TPW - 2, (TPW - 2) % 2).wait()

        @pl.when(valid(TPW - 1))
        def _():
            out_copy(t0 + TPW - 1, (TPW - 1) % 2).wait()

        @pl.when(wid == NW - 1)
        def _():
            pltpu.sync_copy(tailT_hbm, vin0)
            transpose_block(vin0, vout0, 40)
            pltpu.sync_copy(
                vout0.at[pl.ds(0, 40)],
                out_hbm.at[pl.ds(TAIL_TC * D, 40)])

    tail = lax.slice(weT, (0, TAIL_TC * 128), (D, V))
    tailT = jnp.pad(tail, ((0, 0), (0, 128 - TAIL_W)))
    return k(weT, tailT)


def _tc_outformat(x):
    BB = 128
    LP = L // 2

    def body(in_ref, out_ref):
        x3 = in_ref[...].reshape(BB, LP, 2 * D)
        for lp in range(LP):
            zt = x3[:, lp, :].T
            out_ref[2 * lp, :, :] = zt[0:D, :]
            out_ref[2 * lp + 1, :, :] = zt[D:2 * D, :]

    return pl.pallas_call(
        body,
        grid=(B // BB,),
        in_specs=[pl.BlockSpec((BB * LP, 2 * D), lambda g: (g, 0))],
        out_specs=pl.BlockSpec((L, D, BB), lambda g: (0, 0, g)),
        out_shape=jax.ShapeDtypeStruct((L, D, B), jnp.float32),
    )(x)


def _sc_embed(we, idx):
    mesh = plsc.VectorSubcoreMesh(core_axis_name="c", subcore_axis_name="s")

    @functools.partial(
        pl.kernel,
        mesh=mesh,
        out_type=jax.ShapeDtypeStruct((B * L // 2, 2 * D), jnp.float32),
        scratch_types=[
            pltpu.VMEM((B_PER_W, L * K), jnp.int32),
            pltpu.VMEM((L * K, D), jnp.float32),
            pltpu.VMEM((L * K, D), jnp.float32),
            pltpu.VMEM((L // 2, 2 * D), jnp.float32),
            pltpu.VMEM((L // 2, 2 * D), jnp.float32),
            pltpu.SemaphoreType.DMA,
            pltpu.SemaphoreType.DMA,
            pltpu.SemaphoreType.DMA,
            pltpu.SemaphoreType.DMA,
        ],
        compiler_params=pltpu.CompilerParams(use_tc_tiling_on_sc=False),
    )
    def k(we_hbm, idx_hbm, out_hbm, idx_all, rows0, rows1, outv0, outv1,
          gsem0, gsem1, wsem0, wsem1):
        rows = (rows0, rows1)
        outv = (outv0, outv1)
        gsem = (gsem0, gsem1)
        wsem = (wsem0, wsem1)

        wid = lax.axis_index("s") * NC + lax.axis_index("c")
        base = wid * B_PER_W

        pltpu.sync_copy(idx_hbm.at[pl.ds(base, B_PER_W)], idx_all)

        def gather_copy(cc, b):
            return pltpu.make_async_copy(
                we_hbm.at[idx_all.at[cc]], rows[b], gsem[b])

        def out_copy(cc, b):
            return pltpu.make_async_copy(
                outv[b], out_hbm.at[pl.ds((base + cc) * (L // 2), L // 2)],
                wsem[b])

        gather_copy(0, 0).start()

        def step(cc, b):
            @pl.when(cc + 1 < B_PER_W)
            def _():
                gather_copy(cc + 1, 1 - b).start()

            gather_copy(cc, b).wait()

            @pl.when(cc >= 2)
            def _():
                out_copy(cc - 2, b).wait()

            rv = rows[b]
            ov = outv[b]

            def row_body(i2, _):
                for par in range(2):
                    r0 = K * (2 * i2 + par)
                    for v in range(D // 16):
                        so = pl.ds(par * D + v * 16, 16)
                        sr = pl.ds(v * 16, 16)
                        ov[i2, so] = rv[r0, sr] + rv[r0 + 1, sr] + rv[r0 + 2, sr]
                return 0

            lax.fori_loop(0, L // 2, row_body, 0)
            out_copy(cc, b).start()

        def pair_body(g, _):
            step(2 * g, 0)
            step(2 * g + 1, 1)
            return 0

        lax.fori_loop(0, B_PER_W // 2, pair_body, 0)
        out_copy(B_PER_W - 2, 0).wait()
        out_copy(B_PER_W - 1, 1).wait()

    return k(we, idx)


def kernel(inputs, we):
    we_lin = _sc_detile(we.T).reshape(2 * TCOLS * D, D)
    idx = inputs.astype(jnp.int32).reshape(B, L * K)
    x = _sc_embed(we_lin, idx)
    ot = _tc_outformat(x)
    return jnp.transpose(ot, (2, 0, 1))


# speedup vs baseline: 2.1016x; 2.1016x over previous
import functools

import jax
import jax.numpy as jnp
from jax import lax
from jax.experimental import pallas as pl
from jax.experimental.pallas import tpu as pltpu
from jax.experimental.pallas import tpu_sc as plsc

B, L, K = 1024, 200, 3
D = 64
V = 1000518
NC, NS = 2, 16
NW = NC * NS
B_PER_W = B // NW


def _tc_outformat(x):
    BB = 128
    LP = L // 2

    def body(in_ref, out_ref):
        x3 = in_ref[...].reshape(BB, LP, 2 * D)
        for lp in range(LP):
            zt = x3[:, lp, :].T
            out_ref[2 * lp, :, :] = zt[0:D, :]
            out_ref[2 * lp + 1, :, :] = zt[D:2 * D, :]

    return pl.pallas_call(
        body,
        grid=(B // BB,),
        in_specs=[pl.BlockSpec((BB * LP, 2 * D), lambda g: (g, 0))],
        out_specs=pl.BlockSpec((L, D, BB), lambda g: (0, 0, g)),
        out_shape=jax.ShapeDtypeStruct((L, D, B), jnp.float32),
    )(x)


def _sc_embed(we, idx):
    mesh = plsc.VectorSubcoreMesh(core_axis_name="c", subcore_axis_name="s")

    @functools.partial(
        pl.kernel,
        mesh=mesh,
        out_type=jax.ShapeDtypeStruct((B * L // 2, 2 * D), jnp.float32),
        scratch_types=[
            pltpu.VMEM((B_PER_W, L * K), jnp.int32),
            pltpu.VMEM((L * K, D), jnp.float32),
            pltpu.VMEM((L * K, D), jnp.float32),
            pltpu.VMEM((L // 2, 2 * D), jnp.float32),
            pltpu.VMEM((L // 2, 2 * D), jnp.float32),
            pltpu.SemaphoreType.DMA,
            pltpu.SemaphoreType.DMA,
            pltpu.SemaphoreType.DMA,
            pltpu.SemaphoreType.DMA,
        ],
        compiler_params=pltpu.CompilerParams(use_tc_tiling_on_sc=False),
    )
    def k(we_hbm, idx_hbm, out_hbm, idx_all, rows0, rows1, outv0, outv1,
          gsem0, gsem1, wsem0, wsem1):
        rows = (rows0, rows1)
        outv = (outv0, outv1)
        gsem = (gsem0, gsem1)
        wsem = (wsem0, wsem1)

        wid = lax.axis_index("s") * NC + lax.axis_index("c")
        base = wid * B_PER_W

        pltpu.sync_copy(idx_hbm.at[pl.ds(base, B_PER_W)], idx_all)

        def gather_copy(cc, b):
            return pltpu.make_async_copy(
                we_hbm.at[idx_all.at[cc]], rows[b], gsem[b])

        def out_copy(cc, b):
            return pltpu.make_async_copy(
                outv[b], out_hbm.at[pl.ds((base + cc) * (L // 2), L // 2)],
                wsem[b])

        gather_copy(0, 0).start()

        def step(cc, b):
            @pl.when(cc + 1 < B_PER_W)
            def _():
                gather_copy(cc + 1, 1 - b).start()

            gather_copy(cc, b).wait()

            @pl.when(cc >= 2)
            def _():
                out_copy(cc - 2, b).wait()

            rv = rows[b]
            ov = outv[b]

            def row_body(i2, _):
                for par in range(2):
                    r0 = K * (2 * i2 + par)
                    for v in range(D // 16):
                        so = pl.ds(par * D + v * 16, 16)
                        sr = pl.ds(v * 16, 16)
                        ov[i2, so] = rv[r0, sr] + rv[r0 + 1, sr] + rv[r0 + 2, sr]
                return 0

            lax.fori_loop(0, L // 2, row_body, 0)
            out_copy(cc, b).start()

        def pair_body(g, _):
            step(2 * g, 0)
            step(2 * g + 1, 1)
            return 0

        lax.fori_loop(0, B_PER_W // 2, pair_body, 0)
        out_copy(B_PER_W - 2, 0).wait()
        out_copy(B_PER_W - 1, 1).wait()

    return k(we, idx)


def kernel(inputs, we):
    idx = inputs.astype(jnp.int32).reshape(B, L * K)
    x = _sc_embed(we, idx)
    ot = _tc_outformat(x)
    return jnp.transpose(ot, (2, 0, 1))
